# baseline (speedup 1.0000x reference)
import jax
import jax.numpy as jnp
from jax.experimental import pallas as pl
from jax.experimental.pallas import tpu as pltpu

_N, _P, _S = 16, 4096, 64
_NBUF = 4
_NB = 2
_C = _N // _NB


def _tent_body(d_ref, s_ref, o_hbm, buf, sems):
    c = pl.program_id(0)
    b = jax.lax.rem(c, _NBUF)
    n0 = c * _NB

    @pl.when(c >= _NBUF)
    def _wait_prev():
        pltpu.make_async_copy(
            buf.at[b], o_hbm.at[pl.ds((c - _NBUF) * _NB, _NB)], sems.at[b]
        ).wait()

    d = d_ref[pl.ds(n0, _NB)]
    x = d[:, 0:1, :]
    y = d[:, 1:2, :]
    m = 0.5 * (x + y)
    h = 0.5 * (y - x)
    sam = s_ref[...].reshape(1, _S, 1)
    buf[b] = jnp.maximum(h - jnp.abs(sam - m), 0.0)
    pltpu.make_async_copy(
        buf.at[b], o_hbm.at[pl.ds(n0, _NB)], sems.at[b]
    ).start()

    @pl.when(c == _C - 1)
    def _drain():
        for k in range(_NBUF):
            cp = _C - _NBUF + k
            pltpu.make_async_copy(
                buf.at[cp % _NBUF],
                o_hbm.at[pl.ds(cp * _NB, _NB)],
                sems.at[cp % _NBUF],
            ).wait()


def kernel(diagrams, samples):
    dt = jnp.transpose(diagrams, (0, 2, 1))
    out_t = pl.pallas_call(
        _tent_body,
        grid=(_C,),
        in_specs=[
            pl.BlockSpec((_N, 2, _P), lambda i: (0, 0, 0)),
            pl.BlockSpec((_S,), lambda i: (0,)),
        ],
        out_specs=pl.BlockSpec(memory_space=pl.ANY),
        out_shape=jax.ShapeDtypeStruct((_N, _S, _P), jnp.float32),
        scratch_shapes=[
            pltpu.VMEM((_NBUF, _NB, _S, _P), jnp.float32),
            pltpu.SemaphoreType.DMA((_NBUF,)),
        ],
    )(dt, samples)
    return jnp.transpose(out_t, (0, 2, 1))


# speedup vs baseline: 1.0810x; 1.0810x over previous
import jax
import jax.numpy as jnp
from jax.experimental import pallas as pl
from jax.experimental.pallas import tpu as pltpu

_N, _P, _S = 16, 4096, 64
_NBUF = 4


def _tent_body(d_ref, s_ref, o_hbm, buf, sems):
    n = pl.program_id(0)
    b = jax.lax.rem(n, _NBUF)
    sam = s_ref[...].reshape(_S, 1)

    @pl.when(n >= _NBUF)
    def _wait_prev():
        pltpu.make_async_copy(buf.at[b], o_hbm.at[n - _NBUF], sems.at[b]).wait()

    d = d_ref[n]
    x = d[0:1, :]
    y = d[1:2, :]
    m = 0.5 * (x + y)
    h = 0.5 * (y - x)
    buf[b] = jnp.maximum(h - jnp.abs(sam - m), 0.0)
    pltpu.make_async_copy(buf.at[b], o_hbm.at[n], sems.at[b]).start()

    @pl.when(n == _N - 1)
    def _drain():
        for k in range(_NBUF):
            pltpu.make_async_copy(
                buf.at[k], o_hbm.at[_N - _NBUF + k], sems.at[k]
            ).wait()


def kernel(diagrams, samples):
    dt = jnp.transpose(diagrams, (0, 2, 1))
    out_t = pl.pallas_call(
        _tent_body,
        grid=(_N,),
        in_specs=[
            pl.BlockSpec((_N, 2, _P), lambda i: (0, 0, 0)),
            pl.BlockSpec((_S,), lambda i: (0,)),
        ],
        out_specs=pl.BlockSpec(memory_space=pl.ANY),
        out_shape=jax.ShapeDtypeStruct((_N, _S, _P), jnp.float32),
        scratch_shapes=[
            pltpu.VMEM((_NBUF, _S, _P), jnp.float32),
            pltpu.SemaphoreType.DMA((_NBUF,)),
        ],
    )(dt, samples)
    return jnp.transpose(out_t, (0, 2, 1))
